# baseline (speedup 1.0000x reference)
import functools

import jax
import jax.numpy as jnp
from jax import lax
from jax.experimental import pallas as pl
from jax.experimental.pallas import tpu as pltpu
from jax.experimental.pallas import tpu_sc as plsc

V = 10000
D = 128
B = 320000
L = 16
NC = 2
NS = 16
NW = NC * NS
BPW = B // NW
C = 40
N = BPW // C
NBUF = 4
UR = 4

_mesh = plsc.VectorSubcoreMesh(core_axis_name="c", subcore_axis_name="s")

_scratch = (
    [pltpu.VMEM((2 * C,), jnp.int32) for _ in range(2)]
    + [pltpu.VMEM((2 * C, D), jnp.float32) for _ in range(NBUF)]
    + [pltpu.VMEM_SHARED((V, D), jnp.float32)]
    + [pltpu.SemaphoreType.DMA for _ in range(2 + 2 * NBUF)]
)


@functools.partial(
    pl.kernel,
    mesh=_mesh,
    out_type=jax.ShapeDtypeStruct((B, D), jnp.float32),
    scratch_types=_scratch,
)
def _product_tuple(x_hbm, idxcat_hbm, out_hbm, *scr):
    islot = scr[0:2]
    rows = scr[2:2 + NBUF]
    xs = scr[2 + NBUF]
    isem = scr[3 + NBUF:5 + NBUF]
    gsem = scr[5 + NBUF:5 + 2 * NBUF]
    wsem = scr[5 + 2 * NBUF:5 + 3 * NBUF]

    sid = lax.axis_index("s")
    wid = sid * NC + lax.axis_index("c")
    base = pl.multiple_of(wid * BPW, 8)
    cbase = wid * N

    rows_per_sub = 624
    pltpu.sync_copy(x_hbm.at[pl.ds(sid * rows_per_sub, rows_per_sub)],
                    xs.at[pl.ds(sid * rows_per_sub, rows_per_sub)])

    @pl.when(sid == 0)
    def _stage_tail():
        tail = NS * rows_per_sub
        pltpu.sync_copy(x_hbm.at[pl.ds(tail, V - tail)],
                        xs.at[pl.ds(tail, V - tail)])

    plsc.subcore_barrier()

    def issue_idx(c, p):
        off = pl.multiple_of((cbase + c) * 2 * C, 8)
        pltpu.async_copy(idxcat_hbm.at[pl.ds(off, 2 * C)], islot[p], isem[p])

    def wait_idx(p):
        pltpu.make_async_copy(idxcat_hbm.at[pl.ds(0, 2 * C)], islot[p], isem[p]).wait()

    def issue_gather(p, b):
        pltpu.async_copy(xs.at[islot[p]], rows[b], gsem[b])

    def wait_gather(b):
        pltpu.make_async_copy(xs.at[islot[0]], rows[b], gsem[b]).wait()

    def compute(b):
        r = rows[b]

        def row_body(t, carry):
            for u in range(UR):
                rr = t * UR + u
                for j in range(D // L):
                    s = pl.ds(j * L, L)
                    r[rr, s] = r[rr, s] * r[C + rr, s]
            return carry

        lax.fori_loop(0, C // UR, row_body, 0)

    def issue_wb(c, b):
        off = pl.multiple_of(base + c * C, 8)
        pltpu.async_copy(rows[b].at[pl.ds(0, C)], out_hbm.at[pl.ds(off, C)], wsem[b])

    def wait_wb(b):
        pltpu.make_async_copy(rows[b].at[pl.ds(0, C)], out_hbm.at[pl.ds(0, C)], wsem[b]).wait()

    def step(c, b, p, has_next=True, idx_ahead=True, drain_wb=True):
        wait_gather(b)
        if idx_ahead:
            issue_idx(c + 2, p)
        if has_next:
            wait_idx(1 - p)
            if drain_wb:
                wait_wb((b + 1) % NBUF)
            issue_gather(1 - p, (b + 1) % NBUF)
        compute(b)
        issue_wb(c, b)

    issue_idx(0, 0)
    issue_idx(1, 1)
    wait_idx(0)
    issue_gather(0, 0)

    for c in range(NBUF):
        step(c, c % NBUF, c % 2, drain_wb=(c >= NBUF - 1))

    RSTEADY = (N - NBUF - 2) // 4

    def steady(i, carry):
        c0 = NBUF + i * 4
        for j in range(4):
            step(c0 + j, j, j % 2)
        return carry

    lax.fori_loop(0, RSTEADY, steady, 0)

    for c in range(NBUF + 4 * RSTEADY, N):
        step(c, c % NBUF, c % 2,
             has_next=(c + 1 <= N - 1),
             idx_ahead=(c + 2 <= N - 1))

    for b in range(NBUF):
        wait_wb(b)


def _prep_indices(tuples_coo):
    blocks = jnp.stack(
        [tuples_coo[0].reshape(B // C, C), tuples_coo[1].reshape(B // C, C)],
        axis=1)
    return blocks.reshape(2 * B)


def kernel(X, adj_t, tuples_coo):
    del adj_t
    return _product_tuple(X, _prep_indices(tuples_coo))


# speedup vs baseline: 1.3721x; 1.3721x over previous
import functools

import jax
import jax.numpy as jnp
from jax import lax
from jax.experimental import pallas as pl
from jax.experimental.pallas import tpu as pltpu
from jax.experimental.pallas import tpu_sc as plsc

V = 10000
D = 128
B = 320000
L = 16
NC = 2
NS = 16
NW = NC * NS
BPW = B // NW
C = 40
N = BPW // C
NBUF = 4
GA = 2
UR = 4

_mesh = plsc.VectorSubcoreMesh(core_axis_name="c", subcore_axis_name="s")

_scratch = (
    [pltpu.VMEM((C,), jnp.int32) for _ in range(2 * NBUF)]
    + [pltpu.VMEM((2, C, D), jnp.float32) for _ in range(NBUF)]
    + [pltpu.VMEM_SHARED((V, D), jnp.float32)]
    + [pltpu.SemaphoreType.DMA for _ in range(3 * NBUF)]
)


@functools.partial(
    pl.kernel,
    mesh=_mesh,
    out_type=jax.ShapeDtypeStruct((B, D), jnp.float32),
    scratch_types=_scratch,
)
def _product_tuple(x_hbm, idx0_hbm, idx1_hbm, out_hbm, *scr):
    islot = tuple((scr[2 * q], scr[2 * q + 1]) for q in range(NBUF))
    rows = scr[2 * NBUF:3 * NBUF]
    xs = scr[3 * NBUF]
    isem = scr[3 * NBUF + 1:3 * NBUF + 1 + NBUF]
    gsem = scr[3 * NBUF + 1 + NBUF:3 * NBUF + 1 + 2 * NBUF]
    wsem = scr[3 * NBUF + 1 + 2 * NBUF:3 * NBUF + 1 + 3 * NBUF]

    sid = lax.axis_index("s")
    wid = sid * NC + lax.axis_index("c")
    base = pl.multiple_of(wid * BPW, 8)

    rows_per_sub = 624
    pltpu.sync_copy(x_hbm.at[pl.ds(sid * rows_per_sub, rows_per_sub)],
                    xs.at[pl.ds(sid * rows_per_sub, rows_per_sub)])

    @pl.when(sid == 0)
    def _stage_tail():
        tail = NS * rows_per_sub
        pltpu.sync_copy(x_hbm.at[pl.ds(tail, V - tail)],
                        xs.at[pl.ds(tail, V - tail)])

    plsc.subcore_barrier()

    def off_of(c):
        return pl.multiple_of(base + c * C, 8)

    def issue_idx(c, q):
        off = off_of(c)
        pltpu.async_copy(idx0_hbm.at[pl.ds(off, C)], islot[q][0], isem[q])
        pltpu.async_copy(idx1_hbm.at[pl.ds(off, C)], islot[q][1], isem[q])

    def wait_idx(q):
        pltpu.make_async_copy(idx0_hbm.at[pl.ds(0, C)], islot[q][0], isem[q]).wait()
        pltpu.make_async_copy(idx1_hbm.at[pl.ds(0, C)], islot[q][1], isem[q]).wait()

    def issue_gather(q, b):
        pltpu.async_copy(xs.at[islot[q][0]], rows[b].at[0], gsem[b])
        pltpu.async_copy(xs.at[islot[q][1]], rows[b].at[1], gsem[b])

    def wait_gather(b):
        pltpu.make_async_copy(xs.at[islot[0][0]], rows[b].at[0], gsem[b]).wait()
        pltpu.make_async_copy(xs.at[islot[0][1]], rows[b].at[1], gsem[b]).wait()

    def compute(b):
        r = rows[b]

        def row_body(t, carry):
            for u in range(UR):
                rr = t * UR + u
                for j in range(D // L):
                    s = pl.ds(j * L, L)
                    r[0, rr, s] = r[0, rr, s] * r[1, rr, s]
            return carry

        lax.fori_loop(0, C // UR, row_body, 0)

    def issue_wb(c, b):
        pltpu.async_copy(rows[b].at[0], out_hbm.at[pl.ds(off_of(c), C)], wsem[b])

    def wait_wb(b):
        pltpu.make_async_copy(rows[b].at[0], out_hbm.at[pl.ds(0, C)], wsem[b]).wait()

    def step(c, b, do_idx=True, do_gather=True, drain_wb=True):
        wait_gather(b)
        if do_idx:
            issue_idx(c + NBUF, b)
        if do_gather:
            wait_idx((b + GA) % NBUF)
            if drain_wb:
                wait_wb((b + GA) % NBUF)
            issue_gather((b + GA) % NBUF, (b + GA) % NBUF)
        compute(b)
        issue_wb(c, b)

    for q in range(NBUF):
        issue_idx(q, q)
    for c in range(GA):
        wait_idx(c)
        issue_gather(c, c)

    for c in range(NBUF):
        step(c, c % NBUF, drain_wb=(c >= GA))

    RSTEADY = (N - NBUF - NBUF - GA) // 4

    def steady(i, carry):
        c0 = NBUF + i * 4
        for j in range(4):
            step(c0 + j, j)
        return carry

    lax.fori_loop(0, RSTEADY, steady, 0)

    for c in range(NBUF + 4 * RSTEADY, N):
        step(c, c % NBUF,
             do_idx=(c + NBUF <= N - 1),
             do_gather=(c + GA <= N - 1))

    for b in range(NBUF):
        wait_wb(b)


def kernel(X, adj_t, tuples_coo):
    del adj_t
    return _product_tuple(X, tuples_coo[0], tuples_coo[1])


# speedup vs baseline: 1.3736x; 1.0011x over previous
import functools

import jax
import jax.numpy as jnp
from jax import lax
from jax.experimental import pallas as pl
from jax.experimental.pallas import tpu as pltpu
from jax.experimental.pallas import tpu_sc as plsc

V = 10000
D = 128
B = 320000
L = 16
NC = 2
NS = 16
NW = NC * NS
BPW = B // NW
C = 40
N = BPW // C
NBUF = 4
GA = 2
UR = 4

_mesh = plsc.VectorSubcoreMesh(core_axis_name="c", subcore_axis_name="s")

_scratch = (
    [pltpu.VMEM((C,), jnp.int32) for _ in range(2 * NBUF)]
    + [pltpu.VMEM((2, C, D), jnp.float32) for _ in range(NBUF)]
    + [pltpu.VMEM_SHARED((V, D), jnp.float32)]
    + [pltpu.SemaphoreType.DMA for _ in range(3 * NBUF)]
)


@functools.partial(
    pl.kernel,
    mesh=_mesh,
    out_type=jax.ShapeDtypeStruct((B, D), jnp.float32),
    scratch_types=_scratch,
)
def _product_tuple(x_hbm, idx0_hbm, idx1_hbm, out_hbm, *scr):
    islot = tuple((scr[2 * q], scr[2 * q + 1]) for q in range(NBUF))
    rows = scr[2 * NBUF:3 * NBUF]
    xs = scr[3 * NBUF]
    isem = scr[3 * NBUF + 1:3 * NBUF + 1 + NBUF]
    gsem = scr[3 * NBUF + 1 + NBUF:3 * NBUF + 1 + 2 * NBUF]
    wsem = scr[3 * NBUF + 1 + 2 * NBUF:3 * NBUF + 1 + 3 * NBUF]

    sid = lax.axis_index("s")
    wid = sid * NC + lax.axis_index("c")
    base = pl.multiple_of(wid * BPW, 8)

    rows_per_sub = 624
    pltpu.sync_copy(x_hbm.at[pl.ds(sid * rows_per_sub, rows_per_sub)],
                    xs.at[pl.ds(sid * rows_per_sub, rows_per_sub)])

    @pl.when(sid == 0)
    def _stage_tail():
        tail = NS * rows_per_sub
        pltpu.sync_copy(x_hbm.at[pl.ds(tail, V - tail)],
                        xs.at[pl.ds(tail, V - tail)])

    plsc.subcore_barrier()

    def off_of(c):
        return pl.multiple_of(base + c * C, 8)

    def issue_idx(c, q):
        off = off_of(c)
        pltpu.async_copy(idx0_hbm.at[pl.ds(off, C)], islot[q][0], isem[q])
        pltpu.async_copy(idx1_hbm.at[pl.ds(off, C)], islot[q][1], isem[q])

    def wait_idx(q):
        pltpu.make_async_copy(idx0_hbm.at[pl.ds(0, C)], islot[q][0], isem[q]).wait()
        pltpu.make_async_copy(idx1_hbm.at[pl.ds(0, C)], islot[q][1], isem[q]).wait()

    _SPLITS = ((0, 24), (24, 16))

    def issue_gather(q, b):
        for k in range(2):
            for o, n in _SPLITS:
                pltpu.async_copy(xs.at[islot[q][k].at[pl.ds(o, n)]],
                                 rows[b].at[k, pl.ds(o, n)], gsem[b])

    def wait_gather(b):
        for k in range(2):
            for o, n in _SPLITS:
                pltpu.make_async_copy(xs.at[islot[0][k].at[pl.ds(o, n)]],
                                      rows[b].at[k, pl.ds(o, n)], gsem[b]).wait()

    def compute(b):
        r = rows[b]

        def row_body(t, carry):
            for u in range(UR):
                rr = t * UR + u
                for j in range(D // L):
                    s = pl.ds(j * L, L)
                    r[0, rr, s] = r[0, rr, s] * r[1, rr, s]
            return carry

        lax.fori_loop(0, C // UR, row_body, 0)

    def issue_wb(c, b):
        pltpu.async_copy(rows[b].at[0], out_hbm.at[pl.ds(off_of(c), C)], wsem[b])

    def wait_wb(b):
        pltpu.make_async_copy(rows[b].at[0], out_hbm.at[pl.ds(0, C)], wsem[b]).wait()

    def step(c, b, do_idx=True, do_gather=True, drain_wb=True):
        wait_gather(b)
        if do_idx:
            issue_idx(c + NBUF, b)
        if do_gather:
            wait_idx((b + GA) % NBUF)
            if drain_wb:
                wait_wb((b + GA) % NBUF)
            issue_gather((b + GA) % NBUF, (b + GA) % NBUF)
        compute(b)
        issue_wb(c, b)

    for q in range(NBUF):
        issue_idx(q, q)
    for c in range(GA):
        wait_idx(c)
        issue_gather(c, c)

    for c in range(NBUF):
        step(c, c % NBUF, drain_wb=(c >= GA))

    RSTEADY = (N - NBUF - NBUF - GA) // 4

    def steady(i, carry):
        c0 = NBUF + i * 4
        for j in range(4):
            step(c0 + j, j)
        return carry

    lax.fori_loop(0, RSTEADY, steady, 0)

    for c in range(NBUF + 4 * RSTEADY, N):
        step(c, c % NBUF,
             do_idx=(c + NBUF <= N - 1),
             do_gather=(c + GA <= N - 1))

    for b in range(NBUF):
        wait_wb(b)


def kernel(X, adj_t, tuples_coo):
    del adj_t
    return _product_tuple(X, tuples_coo[0], tuples_coo[1])
